# baseline (speedup 1.0000x reference)
import functools

import jax
import jax.numpy as jnp
from jax import lax
from jax.experimental import pallas as pl
from jax.experimental.pallas import tpu as pltpu
from jax.experimental.pallas import tpu_sc as plsc

_NSEG = 16
_SEG = 1024
_D = 512
_L = 16
_QCOL = 128
_CHUNK = 256
_REP = 128

_SCSEG = 6
_TCSEG = _NSEG - _SCSEG
_NITEM = _SCSEG * (_D // _QCOL)

_TBLK = 512


def _sc_body(x_hbm, out_hbm, in0, in1, ob, sem_in, sem_out):
    wid = lax.axis_index("c") * 16 + lax.axis_index("s")

    @pl.when(wid < _NITEM)
    def _():
        seg = wid // 4
        col0 = (wid % 4) * _QCOL
        in_bufs = (in0, in1)
        inv = jnp.full((_L,), 1.0 / _SEG, dtype=jnp.float32)
        n_chunks = _SEG // _CHUNK

        def in_copy(chunk, buf):
            return pltpu.make_async_copy(
                x_hbm.at[pl.ds(seg * _SEG + chunk * _CHUNK, _CHUNK),
                         pl.ds(col0, _QCOL)],
                buf, sem_in)

        in_copy(0, in_bufs[0]).start()

        accs = tuple(jnp.zeros((_L,), jnp.float32) for _ in range(8))
        for chunk in range(n_chunks):
            buf = in_bufs[chunk % 2]
            in_copy(chunk, buf).wait()
            if chunk + 1 < n_chunks:
                in_copy(chunk + 1, in_bufs[(chunk + 1) % 2]).start()

            def red_step(t, a):
                r0 = t * 8
                for r in range(8):
                    a = tuple(
                        a[g] + buf[r0 + r, pl.ds(g * _L, _L)]
                        for g in range(8)
                    )
                return a

            accs = lax.fori_loop(0, _CHUNK // 8, red_step, accs)

        means = tuple(a * inv for a in accs)

        def rep_step(i, _):
            for g in range(8):
                ob[i, pl.ds(g * _L, _L)] = means[g]
            return 0

        lax.fori_loop(0, _REP, rep_step, 0)

        for r in range(_SEG // _REP):
            pltpu.make_async_copy(
                ob,
                out_hbm.at[pl.ds(seg * _SEG + r * _REP, _REP),
                           pl.ds(col0, _QCOL)],
                sem_out).start()

        for _i in range(_SEG // _REP):
            pltpu.make_async_copy(
                ob, out_hbm.at[pl.ds(0, _REP), pl.ds(0, _QCOL)], sem_out
            ).wait()


def _tc_mean_body(x_ref, m_ref):
    s = pl.program_id(0)
    c = pl.program_id(1)

    @pl.when((s == 0) & (c == 0))
    def _():
        m_ref[...] = jnp.zeros_like(m_ref)

    scale = jnp.where(c == _SEG // _TBLK - 1, 1.0 / _SEG, 1.0)
    m_ref[pl.ds(s, 1), :] = scale * (
        m_ref[pl.ds(s, 1), :] + jnp.sum(x_ref[...], axis=0, keepdims=True))


def _tc_bcast_body(m_ref, o_ref):
    s = pl.program_id(0)
    o_ref[...] = jnp.broadcast_to(m_ref[pl.ds(s, 1), :], o_ref.shape)


def kernel(embedded_site_features):
    x = embedded_site_features

    mesh = plsc.VectorSubcoreMesh(core_axis_name="c", subcore_axis_name="s")
    sc_run = functools.partial(
        pl.kernel,
        mesh=mesh,
        out_type=jax.ShapeDtypeStruct((_SCSEG * _SEG, _D), jnp.float32),
        scratch_types=[
            pltpu.VMEM((_CHUNK, _QCOL), jnp.float32),
            pltpu.VMEM((_CHUNK, _QCOL), jnp.float32),
            pltpu.VMEM((_REP, _QCOL), jnp.float32),
            pltpu.SemaphoreType.DMA,
            pltpu.SemaphoreType.DMA,
        ],
        compiler_params=pltpu.CompilerParams(use_tc_tiling_on_sc=True),
    )(_sc_body)
    sc_out = sc_run(x)

    n_tblk = _SEG // _TBLK
    tc_means = pl.pallas_call(
        _tc_mean_body,
        grid=(_TCSEG, n_tblk),
        in_specs=[pl.BlockSpec(
            (_TBLK, _D),
            lambda s, c: (_SCSEG * (_SEG // _TBLK) + s * (_SEG // _TBLK) + c, 0))],
        out_specs=pl.BlockSpec((_TCSEG, _D), lambda s, c: (0, 0)),
        out_shape=jax.ShapeDtypeStruct((_TCSEG, _D), jnp.float32),
        compiler_params=pltpu.CompilerParams(
            dimension_semantics=("arbitrary", "arbitrary")),
    )(x)

    tc_out = pl.pallas_call(
        _tc_bcast_body,
        grid=(_TCSEG, n_tblk),
        in_specs=[pl.BlockSpec((_TCSEG, _D), lambda s, c: (0, 0))],
        out_specs=pl.BlockSpec(
            (_TBLK, _D), lambda s, c: (s * (_SEG // _TBLK) + c, 0)),
        out_shape=jax.ShapeDtypeStruct((_TCSEG * _SEG, _D), jnp.float32),
    )(tc_means)

    return jnp.concatenate([sc_out, tc_out], axis=0)


# speedup vs baseline: 1.4512x; 1.4512x over previous
import functools

import jax
import jax.numpy as jnp
from jax import lax
from jax.experimental import pallas as pl
from jax.experimental.pallas import tpu as pltpu
from jax.experimental.pallas import tpu_sc as plsc

_NSEG = 16
_SEG = 1024
_D = 512
_L = 16
_QCOL = 128
_CHUNK = 256

_SCSEG = 6
_TCSEG = _NSEG - _SCSEG
_NITEM = _SCSEG * (_D // _QCOL)

_TBLK = 512


def _sc_body(x_hbm, mean_hbm, in0, in1, ob, sem_in, sem_out):
    wid = lax.axis_index("c") * 16 + lax.axis_index("s")

    @pl.when(wid < _NITEM)
    def _():
        seg = wid // 4
        col0 = (wid % 4) * _QCOL
        in_bufs = (in0, in1)
        inv = jnp.full((_L,), 1.0 / _SEG, dtype=jnp.float32)
        n_chunks = _SEG // _CHUNK

        def in_copy(chunk, buf):
            return pltpu.make_async_copy(
                x_hbm.at[pl.ds(seg * _SEG + chunk * _CHUNK, _CHUNK),
                         pl.ds(col0, _QCOL)],
                buf, sem_in)

        in_copy(0, in_bufs[0]).start()

        accs = tuple(jnp.zeros((_L,), jnp.float32) for _ in range(8))
        for chunk in range(n_chunks):
            buf = in_bufs[chunk % 2]
            in_copy(chunk, buf).wait()
            if chunk + 1 < n_chunks:
                in_copy(chunk + 1, in_bufs[(chunk + 1) % 2]).start()

            def red_step(t, a):
                r0 = t * 8
                for r in range(8):
                    a = tuple(
                        a[g] + buf[r0 + r, pl.ds(g * _L, _L)]
                        for g in range(8)
                    )
                return a

            accs = lax.fori_loop(0, _CHUNK // 8, red_step, accs)

        means = tuple(a * inv for a in accs)

        for r in range(8):
            for g in range(8):
                ob[r, pl.ds(g * _L, _L)] = means[g]
        pltpu.make_async_copy(
            ob,
            mean_hbm.at[pl.ds(seg * 8, 8), pl.ds(col0, _QCOL)],
            sem_out).start()
        pltpu.make_async_copy(
            ob, mean_hbm.at[pl.ds(0, 8), pl.ds(0, _QCOL)], sem_out
        ).wait()


def _tc_mean_body(x_ref, m_ref):
    s = pl.program_id(0)
    c = pl.program_id(1)

    @pl.when((s == 0) & (c == 0))
    def _():
        m_ref[...] = jnp.zeros_like(m_ref)

    scale = jnp.where(c == _SEG // _TBLK - 1, 1.0 / _SEG, 1.0)
    m_ref[pl.ds(s, 1), :] = scale * (
        m_ref[pl.ds(s, 1), :] + jnp.sum(x_ref[...], axis=0, keepdims=True))


def _tc_bcast_body(msc_ref, mtc_ref, o_ref):
    s = pl.program_id(0)

    @pl.when(s < _SCSEG)
    def _():
        o_ref[...] = jnp.broadcast_to(
            msc_ref[pl.ds(s * 8, 1), :], o_ref.shape)

    @pl.when(s >= _SCSEG)
    def _():
        o_ref[...] = jnp.broadcast_to(
            mtc_ref[pl.ds(s - _SCSEG, 1), :], o_ref.shape)


def kernel(embedded_site_features):
    x = embedded_site_features

    mesh = plsc.VectorSubcoreMesh(core_axis_name="c", subcore_axis_name="s")
    sc_run = functools.partial(
        pl.kernel,
        mesh=mesh,
        out_type=jax.ShapeDtypeStruct((_SCSEG * 8, _D), jnp.float32),
        scratch_types=[
            pltpu.VMEM((_CHUNK, _QCOL), jnp.float32),
            pltpu.VMEM((_CHUNK, _QCOL), jnp.float32),
            pltpu.VMEM((8, _QCOL), jnp.float32),
            pltpu.SemaphoreType.DMA,
            pltpu.SemaphoreType.DMA,
        ],
        compiler_params=pltpu.CompilerParams(use_tc_tiling_on_sc=True),
    )(_sc_body)
    sc_means = sc_run(x)

    n_tblk = _SEG // _TBLK
    tc_means = pl.pallas_call(
        _tc_mean_body,
        grid=(_TCSEG, n_tblk),
        in_specs=[pl.BlockSpec(
            (_TBLK, _D),
            lambda s, c: (_SCSEG * (_SEG // _TBLK) + s * (_SEG // _TBLK) + c, 0))],
        out_specs=pl.BlockSpec((_TCSEG, _D), lambda s, c: (0, 0)),
        out_shape=jax.ShapeDtypeStruct((_TCSEG, _D), jnp.float32),
        compiler_params=pltpu.CompilerParams(
            dimension_semantics=("arbitrary", "arbitrary")),
    )(x)

    out = pl.pallas_call(
        _tc_bcast_body,
        grid=(_NSEG, n_tblk),
        in_specs=[
            pl.BlockSpec((_SCSEG * 8, _D), lambda s, c: (0, 0)),
            pl.BlockSpec((_TCSEG, _D), lambda s, c: (0, 0)),
        ],
        out_specs=pl.BlockSpec(
            (_TBLK, _D), lambda s, c: (s * (_SEG // _TBLK) + c, 0)),
        out_shape=jax.ShapeDtypeStruct((_NSEG * _SEG, _D), jnp.float32),
    )(sc_means, tc_means)

    return out
